# Initial kernel scaffold; baseline (speedup 1.0000x reference)
import jax
import jax.numpy as jnp
from jax.experimental import pallas as pl


def kernel(inputs, token_table, pos_table):
    raise NotImplementedError("write your pallas kernel here")


# speedup vs baseline: 1.3914x; 1.3914x over previous
import functools

import jax
import jax.numpy as jnp
from jax import lax
from jax.experimental import pallas as pl
from jax.experimental.pallas import tpu as pltpu
from jax.experimental.pallas import tpu_sc as plsc

NC, NS = 2, 16
NW = NC * NS

B, L, D = 4096, 200, 32
ROWS = B * L
IPR = 100
IDX_ROWS = ROWS // IPR
GPC = 8
CHUNK = GPC * IPR
SEQ_PER_CHUNK = CHUNK // L
RPW = ROWS // NW
CPW = RPW // CHUNK


@functools.partial(
    pl.kernel,
    out_type=jax.ShapeDtypeStruct((ROWS, D), jnp.float32),
    mesh=plsc.VectorSubcoreMesh(core_axis_name="c", subcore_axis_name="s"),
    compiler_params=pltpu.CompilerParams(use_tc_tiling_on_sc=False),
    scratch_types=[
        pltpu.VMEM((GPC, IPR), jnp.int32),
        pltpu.VMEM((CHUNK, D), jnp.float32),
        pltpu.VMEM((L, D), jnp.float32),
        pltpu.SemaphoreType.DMA,
    ],
)
def _sc_embed(idx_hbm, tab_hbm, pos_hbm, out_hbm, idx_v, rows_v, pos_v, gsem):
    wid = lax.axis_index("s") * NC + lax.axis_index("c")
    pltpu.sync_copy(pos_hbm, pos_v)
    idx_base = wid * (RPW // IPR)
    row_base = wid * RPW

    def chunk_fn(c, carry):
        ib = idx_base + c * GPC
        rb = row_base + c * CHUNK
        pltpu.sync_copy(idx_hbm.at[pl.ds(ib, GPC)], idx_v)
        copies = [
            pltpu.async_copy(
                tab_hbm.at[idx_v.at[j]],
                rows_v.at[pl.ds(j * IPR, IPR)],
                gsem,
            )
            for j in range(GPC)
        ]
        for cp in copies:
            cp.wait()

        def add_fn(i, inner):
            p0 = pos_v[i, pl.ds(0, 16)]
            p1 = pos_v[i, pl.ds(16, 16)]
            for s in range(SEQ_PER_CHUNK):
                r = s * L + i
                rows_v[r, pl.ds(0, 16)] = rows_v[r, pl.ds(0, 16)] + p0
                rows_v[r, pl.ds(16, 16)] = rows_v[r, pl.ds(16, 16)] + p1
            return inner

        lax.fori_loop(0, L, add_fn, 0)
        pltpu.sync_copy(rows_v, out_hbm.at[pl.ds(rb, CHUNK)])
        return carry

    lax.fori_loop(0, CPW, chunk_fn, 0)


def kernel(inputs, token_table, pos_table):
    idx2d = inputs.astype(jnp.int32).reshape(IDX_ROWS, IPR)
    out = _sc_embed(idx2d, token_table, pos_table)
    return out.reshape(B, L, D)


# speedup vs baseline: 1.4496x; 1.0419x over previous
import functools

import jax
import jax.numpy as jnp
from jax import lax
from jax.experimental import pallas as pl
from jax.experimental.pallas import tpu as pltpu
from jax.experimental.pallas import tpu_sc as plsc

NC, NS = 2, 16
NW = NC * NS

B, L, D = 4096, 200, 32
TD, TR, TC = D // 8, 8, 128
NU = L


@functools.partial(
    pl.kernel,
    out_type=(
        jax.ShapeDtypeStruct((L, TD, NW, TR, TC), jnp.float32),
        jax.ShapeDtypeStruct((TD, TR, TC), jnp.float32),
    ),
    mesh=plsc.VectorSubcoreMesh(core_axis_name="c", subcore_axis_name="s"),
    compiler_params=pltpu.CompilerParams(
        use_tc_tiling_on_sc=False, needs_layout_passes=False),
    scratch_types=[
        [pltpu.VMEM((1, TC), jnp.int32) for _ in range(2)],
        [pltpu.VMEM((TC, D), jnp.float32) for _ in range(2)],
        [pltpu.VMEM((D, TC), jnp.float32) for _ in range(2)],
        pltpu.VMEM((L, D), jnp.float32),
        [pltpu.SemaphoreType.DMA for _ in range(2)],
        [pltpu.SemaphoreType.DMA for _ in range(2)],
        [pltpu.SemaphoreType.DMA for _ in range(2)],
    ],
)
def _sc_embed(idxT_hbm, tab_hbm, pos_hbm, out5, dummy, idx_vs, rows_vs,
              trans_vs, pos_v, isems, gsems, ssems):
    wid = lax.axis_index("s") * NC + lax.axis_index("c")
    b0 = wid * TC
    pltpu.sync_copy(pos_hbm, pos_v)
    iota = lax.iota(jnp.int32, 16)

    def issue_idx(u, sl):
        pltpu.async_copy(idxT_hbm.at[pl.ds(u, 1), pl.ds(b0, TC)],
                         idx_vs[sl], isems[sl])

    def issue_gather(sl):
        pltpu.async_copy(tab_hbm.at[idx_vs[sl].at[0]], rows_vs[sl], gsems[sl])

    def wait_idx(sl):
        pltpu.make_async_copy(idxT_hbm.at[pl.ds(0, 1), pl.ds(0, TC)],
                              idx_vs[sl], isems[sl]).wait()

    def wait_gather(sl):
        pltpu.make_async_copy(tab_hbm.at[pl.ds(0, TC)],
                              rows_vs[sl], gsems[sl]).wait()

    def wait_store(sl):
        pltpu.make_async_copy(dummy, trans_vs[sl], ssems[sl]).wait()

    def unit(l, sl):
        rows_v = rows_vs[sl]
        trans_v = trans_vs[sl]
        p0 = pos_v[l, pl.ds(0, 16)]
        p1 = pos_v[l, pl.ds(16, 16)]

        def col(c, carry):
            cv = jnp.full((16,), 0, jnp.int32) + c
            v0 = rows_v[c, pl.ds(0, 16)] + p0
            v1 = rows_v[c, pl.ds(16, 16)] + p1
            plsc.store_scatter(trans_v, [iota, cv], v0)
            plsc.store_scatter(trans_v, [iota + 16, cv], v1)
            return carry

        lax.fori_loop(0, TC, col, 0)
        for td in range(TD):
            pltpu.async_copy(trans_v.at[pl.ds(td * TR, TR)],
                             out5.at[l, td, wid], ssems[sl])

    pltpu.sync_copy(idxT_hbm.at[pl.ds(0, 1), pl.ds(b0, TC)], idx_vs[0])
    issue_gather(0)
    issue_idx(1, 1)

    def body(j, carry):
        for p in range(2):
            u = 2 * j + p
            q = 1 - p
            @pl.when(u + 1 < NU)
            def _():
                wait_idx(q)
                issue_gather(q)

            @pl.when(u + 2 < NU)
            def _():
                issue_idx(u + 2, p)

            @pl.when(u >= 2)
            def _():
                wait_store(p)

            wait_gather(p)
            unit(u, p)
        return carry

    lax.fori_loop(0, NU // 2, body, 0)
    wait_store(0)
    wait_store(1)


def kernel(inputs, token_table, pos_table):
    out5, _ = _sc_embed(inputs.T, token_table, pos_table)
    return out5.transpose(2, 4, 0, 1, 3).reshape(B, L, D)
